# Initial kernel scaffold; baseline (speedup 1.0000x reference)
import jax
import jax.numpy as jnp
from jax.experimental import pallas as pl


def kernel(nodes, edges, attention, edge_index):
    raise NotImplementedError("write your pallas kernel here")


# speedup vs baseline: 5.4642x; 5.4642x over previous
import functools

import jax
import jax.numpy as jnp
from jax import lax
from jax.experimental import pallas as pl
from jax.experimental.pallas import tpu as pltpu
from jax.experimental.pallas import tpu_sc as plsc

NC = 2
NS = 16
L = 16
NW = NC * NS
C = 80


def _sc_accumulate(n_pad, e, d, edges, att, idx):
    epw = e // NW
    nd = n_pad // (8 * L)
    assert epw * NW == e and epw % C == 0 and d == 8 * L
    assert n_pad % (NS * C) == 0 and nd <= C and nd % 8 == 0
    n_chunks = epw // C
    rows_pt = n_pad // NS
    assert rows_pt % C == 0
    dexp = 8
    assert nd % dexp == 0 and nd // dexp <= NS

    mesh = plsc.VectorSubcoreMesh(
        core_axis_name="c", subcore_axis_name="s", num_cores=NC, num_subcores=NS
    )

    @functools.partial(
        pl.kernel,
        out_type=(
            jax.ShapeDtypeStruct((NC, n_pad, d), jnp.float32),
            jax.ShapeDtypeStruct((NC, nd, d), jnp.float32),
        ),
        mesh=mesh,
        scratch_types=[
            pltpu.VMEM_SHARED((n_pad, d), jnp.float32),
            pltpu.VMEM_SHARED((nd, d), jnp.float32),
            pltpu.VMEM((C, d), jnp.float32),
            pltpu.VMEM((C,), jnp.float32),
            pltpu.VMEM((nd, d), jnp.float32),
            pltpu.VMEM((C,), jnp.int32),
        ],
    )
    def k(edges_h, att_h, idx_h, acc_o, den_o, acc_s, den_s,
          ebuf, abuf, dloc, ibuf):
        cid = lax.axis_index("c")
        sid = lax.axis_index("s")
        wid = cid * NS + sid

        def zrow(i, _):
            z = jnp.zeros((L,), jnp.float32)
            for r in range(d // L):
                ebuf[i, pl.ds(r * L, L)] = z
            return 0

        lax.fori_loop(0, C, zrow, 0)

        def zdrow(i, _):
            z = jnp.zeros((L,), jnp.float32)
            for r in range(d // L):
                dloc[i, pl.ds(r * L, L)] = z
            return 0

        lax.fori_loop(0, nd, zdrow, 0)
        r0 = sid * rows_pt
        for i in range(rows_pt // C):
            pltpu.sync_copy(ebuf, acc_s.at[pl.ds(r0 + i * C, C)])

        @pl.when(sid == 0)
        def _():
            pltpu.sync_copy(ebuf.at[pl.ds(0, nd)], den_s.at[pl.ds(0, nd)])

        plsc.subcore_barrier()

        lanes = lax.iota(jnp.int32, L)

        def chunk_body(kk, _):
            base = wid * epw + kk * C
            pltpu.sync_copy(edges_h.at[pl.ds(base, C)], ebuf)
            pltpu.sync_copy(att_h.at[pl.ds(base, C)], abuf)
            pltpu.sync_copy(idx_h.at[pl.ds(base, C)], ibuf)

            def group_body(g, _):
                iv = ibuf[pl.ds(g * L, L)]
                rv = lax.shift_right_logical(iv, 7)
                gv = jnp.bitwise_and(lax.shift_right_logical(iv, 4), 7) * L
                lv = jnp.bitwise_and(iv, 15)
                wv = jnp.exp(abuf[pl.ds(g * L, L)])
                for t in range(L):
                    bw = jnp.full((L,), wv[t], jnp.float32)
                    j = g * L + t
                    dv = dloc[rv[t], pl.ds(gv[t], L)]
                    dv = dv + jnp.where(lanes == lv[t], bw, 0.0)
                    dloc[rv[t], pl.ds(gv[t], L)] = dv
                    for r in range(d // L):
                        ebuf[j, pl.ds(r * L, L)] = ebuf[j, pl.ds(r * L, L)] * bw
                return 0

            lax.fori_loop(0, C // L, group_body, 0)
            pltpu.sync_copy(ebuf, acc_s.at[ibuf], add=True)
            return 0

        lax.fori_loop(0, n_chunks, chunk_body, 0)

        for g in range(nd // L):
            ibuf[pl.ds(g * L, L)] = lanes + g * L
        pltpu.sync_copy(dloc, den_s.at[ibuf], add=True)
        plsc.subcore_barrier()

        for i in range(rows_pt // C):
            lo = r0 + i * C
            pltpu.sync_copy(acc_s.at[pl.ds(lo, C)], ebuf)
            pltpu.sync_copy(ebuf, acc_o.at[cid, pl.ds(lo, C)])

        @pl.when(sid < nd // dexp)
        def _():
            dlo = sid * dexp
            pltpu.sync_copy(den_s.at[pl.ds(dlo, dexp)], ebuf.at[pl.ds(0, dexp)])
            pltpu.sync_copy(ebuf.at[pl.ds(0, dexp)],
                            den_o.at[cid, pl.ds(dlo, dexp)])

    return k(edges, att, idx)


def _combine(acc, den, n_pad, d):
    blk = 512

    def body(a_ref, d_ref, o_ref):
        dn = d_ref[0] + d_ref[1]
        dn = jnp.where(dn == 0.0, 1.0, dn)
        o_ref[...] = (a_ref[0] + a_ref[1]) / dn

    return pl.pallas_call(
        body,
        grid=(n_pad // blk,),
        in_specs=[
            pl.BlockSpec((2, blk, d), lambda i: (0, i, 0)),
            pl.BlockSpec((2, blk, 1), lambda i: (0, i, 0)),
        ],
        out_specs=pl.BlockSpec((blk, d), lambda i: (i, 0)),
        out_shape=jax.ShapeDtypeStruct((n_pad, d), jnp.float32),
    )(acc, den)


def kernel(nodes, edges, attention, edge_index):
    n = nodes.shape[0]
    e, d = edges.shape
    n_pad = -(-n // (NS * C)) * (NS * C)
    receive = edge_index[0].astype(jnp.int32)
    att = attention.reshape(e).astype(jnp.float32)
    acc, den_c = _sc_accumulate(n_pad, e, d, edges, att, receive)
    den = den_c.reshape(NC, n_pad, 1)
    out = _combine(acc, den, n_pad, d)
    return out[:n]
